# baseline (speedup 1.0000x reference)
import jax
import jax.numpy as jnp
from jax import lax
from jax.experimental import pallas as pl
from jax.experimental.pallas import tpu as pltpu

VOCAB = 33
EMBED_DIM = 1280
PADDING_IDX = 1
MASK_IDX = 32
LN_EPS = 1e-5
VPAD = 64

B = 32
S = 1024
TBLK = 512
SPB = S // TBLK
NCH = B * S // TBLK
NQ = 4


def _stage_a_body(tokens_ref, table_ref, gamma_ref, beta_ref, n2_ref):
    tok = tokens_ref[...]
    n_nonpad = jnp.sum((tok != PADDING_IDX).astype(jnp.float32), axis=1, keepdims=True)
    n_mask = jnp.sum((tok == MASK_IDX).astype(jnp.float32), axis=1, keepdims=True)
    s = 0.88 * n_nonpad / (n_nonpad - n_mask)

    tab = table_ref[...]
    rid = jax.lax.broadcasted_iota(jnp.int32, (VPAD, EMBED_DIM), 0)
    keep = ((rid != PADDING_IDX) & (rid != MASK_IDX)).astype(jnp.float32)
    tabk = (tab * keep)[None]
    x = tabk * s[:, :, None]
    mean = jnp.mean(x, axis=2, keepdims=True)
    var = jnp.mean((x - mean) * (x - mean), axis=2, keepdims=True)
    inv = jax.lax.rsqrt(var + LN_EPS)
    n = (x - mean) * inv * gamma_ref[...][None] + beta_ref[...][None]

    hi = n.astype(jnp.bfloat16)
    lo = (n - hi.astype(jnp.float32)).astype(jnp.bfloat16)
    n2_ref[...] = jnp.concatenate([hi, lo], axis=1)


def _stage_b_body(trow_ref, n2_ref, out_ref, scr_ref, m0, m1, m2, m3):
    sem = [m0, m1, m2, m3]
    p = pl.program_id(0)
    q = lax.rem(p, NQ)

    t = trow_ref[0]
    v = jax.lax.broadcasted_iota(jnp.int32, (2 * VPAD, TBLK), 0) & (VPAD - 1)
    onehot_t = (t == v).astype(jnp.bfloat16)

    for qq in range(NQ):
        @pl.when(jnp.logical_and(q == qq, p >= NQ))
        def _():
            pltpu.make_async_copy(
                scr_ref.at[pl.ds(qq * TBLK, TBLK), :],
                out_ref.at[pl.ds((p - NQ) * TBLK, TBLK), :], sem[qq]
            ).wait()

    scr_ref[pl.ds(q * TBLK, TBLK), :] = jax.lax.dot_general(
        onehot_t, n2_ref[0],
        (((0,), (0,)), ((), ())),
        preferred_element_type=jnp.float32,
    )

    for qq in range(NQ):
        @pl.when(q == qq)
        def _():
            pltpu.make_async_copy(
                scr_ref.at[pl.ds(qq * TBLK, TBLK), :],
                out_ref.at[pl.ds(p * TBLK, TBLK), :], sem[qq]
            ).start()

    @pl.when(p == NCH - 1)
    def _():
        for c in range(NCH - NQ, NCH):
            pltpu.make_async_copy(
                scr_ref.at[pl.ds((c % NQ) * TBLK, TBLK), :],
                out_ref.at[pl.ds(c * TBLK, TBLK), :], sem[c % NQ]
            ).wait()


def kernel(tokens, chain_ids, embed_table, ln_gamma, ln_beta):
    del chain_ids
    tokens = tokens.astype(jnp.int32)
    table_pad = jnp.zeros((VPAD, EMBED_DIM), jnp.float32).at[:VOCAB].set(embed_table)

    n2 = pl.pallas_call(
        _stage_a_body,
        grid=(1,),
        in_specs=[
            pl.BlockSpec((B, S), lambda i: (0, 0)),
            pl.BlockSpec((VPAD, EMBED_DIM), lambda i: (0, 0)),
            pl.BlockSpec((1, EMBED_DIM), lambda i: (0, 0)),
            pl.BlockSpec((1, EMBED_DIM), lambda i: (0, 0)),
        ],
        out_specs=pl.BlockSpec((B, 2 * VPAD, EMBED_DIM), lambda i: (0, 0, 0)),
        out_shape=jax.ShapeDtypeStruct((B, 2 * VPAD, EMBED_DIM), jnp.bfloat16),
    )(
        tokens,
        table_pad,
        ln_gamma.reshape(1, EMBED_DIM),
        ln_beta.reshape(1, EMBED_DIM),
    )

    out = pl.pallas_call(
        _stage_b_body,
        grid=(NCH,),
        in_specs=[
            pl.BlockSpec((1, 1, TBLK), lambda p: (p, 0, 0)),
            pl.BlockSpec((1, 2 * VPAD, EMBED_DIM), lambda p: (p // SPB, 0, 0)),
        ],
        out_specs=pl.BlockSpec(memory_space=pl.ANY),
        out_shape=jax.ShapeDtypeStruct((B * S, EMBED_DIM), jnp.float32),
        scratch_shapes=[pltpu.VMEM((NQ * TBLK, EMBED_DIM), jnp.float32)]
        + [pltpu.SemaphoreType.DMA] * NQ,
        compiler_params=pltpu.CompilerParams(
            dimension_semantics=("arbitrary",),
        ),
    )(
        tokens.reshape(NCH, 1, TBLK),
        n2,
    )
    return out.reshape(B, S, EMBED_DIM)


# speedup vs baseline: 1.3114x; 1.3114x over previous
import jax
import jax.numpy as jnp
from jax.experimental import pallas as pl
from jax.experimental.pallas import tpu as pltpu

VOCAB = 33
EMBED_DIM = 1280
PADDING_IDX = 1
MASK_IDX = 32
LN_EPS = 1e-5
VPAD = 64

B = 32
S = 1024
RPG = 2
TBLK = RPG * S
K = RPG * 2 * VPAD
NSTEP = B // RPG


def _body(trow_ref, tokens_ref, table_ref, gamma_ref, beta_ref, out_ref, n2_scr):
    p = pl.program_id(0)

    @pl.when(p == 0)
    def _():
        tok = tokens_ref[...]
        n_nonpad = jnp.sum((tok != PADDING_IDX).astype(jnp.float32), axis=1,
                           keepdims=True)
        n_mask = jnp.sum((tok == MASK_IDX).astype(jnp.float32), axis=1,
                         keepdims=True)
        s = 0.88 * n_nonpad / (n_nonpad - n_mask)

        tab = table_ref[...]
        rid = jax.lax.broadcasted_iota(jnp.int32, (VPAD, EMBED_DIM), 0)
        keep = ((rid != PADDING_IDX) & (rid != MASK_IDX)).astype(jnp.float32)
        tabk = (tab * keep)[None]
        x = tabk * s[:, :, None]
        mean = jnp.mean(x, axis=2, keepdims=True)
        var = jnp.mean((x - mean) * (x - mean), axis=2, keepdims=True)
        inv = jax.lax.rsqrt(var + LN_EPS)
        n = (x - mean) * inv * gamma_ref[...][None] + beta_ref[...][None]

        hi = n.astype(jnp.bfloat16)
        lo = (n - hi.astype(jnp.float32)).astype(jnp.bfloat16)
        for b in range(B):
            n2_scr[pl.ds(b * 2 * VPAD, VPAD), :] = hi[b]
            n2_scr[pl.ds(b * 2 * VPAD + VPAD, VPAD), :] = lo[b]

    t = trow_ref[0]
    v = jax.lax.broadcasted_iota(jnp.int32, (K, TBLK), 0)
    i = jax.lax.broadcasted_iota(jnp.int32, (K, TBLK), 1)
    onehot_t = ((t == (v & (VPAD - 1)))
                & ((v >> 7) == (i >> 10))).astype(jnp.bfloat16)
    out_ref[...] = jax.lax.dot_general(
        onehot_t, n2_scr[pl.ds(p * K, K), :],
        (((0,), (0,)), ((), ())),
        preferred_element_type=jnp.float32,
    )


def kernel(tokens, chain_ids, embed_table, ln_gamma, ln_beta):
    del chain_ids
    tokens = tokens.astype(jnp.int32)
    table_pad = jnp.zeros((VPAD, EMBED_DIM), jnp.float32).at[:VOCAB].set(embed_table)

    out = pl.pallas_call(
        _body,
        grid=(NSTEP,),
        in_specs=[
            pl.BlockSpec((1, 1, TBLK), lambda p: (p, 0, 0)),
            pl.BlockSpec((B, S), lambda p: (0, 0)),
            pl.BlockSpec((VPAD, EMBED_DIM), lambda p: (0, 0)),
            pl.BlockSpec((1, EMBED_DIM), lambda p: (0, 0)),
            pl.BlockSpec((1, EMBED_DIM), lambda p: (0, 0)),
        ],
        out_specs=pl.BlockSpec((TBLK, EMBED_DIM), lambda p: (p, 0)),
        out_shape=jax.ShapeDtypeStruct((B * S, EMBED_DIM), jnp.float32),
        scratch_shapes=[pltpu.VMEM((B * 2 * VPAD, EMBED_DIM), jnp.bfloat16)],
        compiler_params=pltpu.CompilerParams(
            dimension_semantics=("arbitrary",),
        ),
    )(
        tokens.reshape(NSTEP, 1, TBLK),
        tokens,
        table_pad,
        ln_gamma.reshape(1, EMBED_DIM),
        ln_beta.reshape(1, EMBED_DIM),
    )
    return out.reshape(B, S, EMBED_DIM)


# speedup vs baseline: 1.3506x; 1.0299x over previous
import jax
import jax.numpy as jnp
from jax.experimental import pallas as pl
from jax.experimental.pallas import tpu as pltpu

VOCAB = 33
EMBED_DIM = 1280
PADDING_IDX = 1
MASK_IDX = 32
LN_EPS = 1e-5
VPAD = 64

B = 32
S = 1024
RPG = 2
TBLK = RPG * S
K = RPG * 2 * VPAD
NSTEP = B // RPG


def _body(tokens_ref, table_ref, gamma_ref, beta_ref, out_ref, n2_scr):
    p = pl.program_id(0)

    @pl.when(p == 0)
    def _():
        tok = tokens_ref[...]
        n_nonpad = jnp.sum((tok != PADDING_IDX).astype(jnp.float32), axis=1,
                           keepdims=True)
        n_mask = jnp.sum((tok == MASK_IDX).astype(jnp.float32), axis=1,
                         keepdims=True)
        s = 0.88 * n_nonpad / (n_nonpad - n_mask)

        tab = table_ref[...]
        rid = jax.lax.broadcasted_iota(jnp.int32, (VPAD, EMBED_DIM), 0)
        keep = ((rid != PADDING_IDX) & (rid != MASK_IDX)).astype(jnp.float32)
        tabk = (tab * keep)[None]
        x = tabk * s[:, :, None]
        mean = jnp.mean(x, axis=2, keepdims=True)
        var = jnp.mean((x - mean) * (x - mean), axis=2, keepdims=True)
        inv = jax.lax.rsqrt(var + LN_EPS)
        n = (x - mean) * inv * gamma_ref[...][None] + beta_ref[...][None]

        hi = n.astype(jnp.bfloat16)
        lo = (n - hi.astype(jnp.float32)).astype(jnp.bfloat16)
        for b in range(B):
            n2_scr[pl.ds(b * 2 * VPAD, VPAD), :] = hi[b]
            n2_scr[pl.ds(b * 2 * VPAD + VPAD, VPAD), :] = lo[b]

    v = jax.lax.broadcasted_iota(jnp.int32, (2 * VPAD, S), 0) & (VPAD - 1)
    for r in range(RPG):
        t = tokens_ref[pl.ds(RPG * p + r, 1), :]
        onehot_t = (t == v).astype(jnp.bfloat16)
        out_ref[pl.ds(r * S, S), :] = jax.lax.dot_general(
            onehot_t, n2_scr[pl.ds((RPG * p + r) * 2 * VPAD, 2 * VPAD), :],
            (((0,), (0,)), ((), ())),
            preferred_element_type=jnp.float32,
        )


def kernel(tokens, chain_ids, embed_table, ln_gamma, ln_beta):
    del chain_ids
    tokens = tokens.astype(jnp.int32)
    table_pad = jnp.zeros((VPAD, EMBED_DIM), jnp.float32).at[:VOCAB].set(embed_table)

    out = pl.pallas_call(
        _body,
        grid=(NSTEP,),
        in_specs=[
            pl.BlockSpec((B, S), lambda p: (0, 0)),
            pl.BlockSpec((VPAD, EMBED_DIM), lambda p: (0, 0)),
            pl.BlockSpec((1, EMBED_DIM), lambda p: (0, 0)),
            pl.BlockSpec((1, EMBED_DIM), lambda p: (0, 0)),
        ],
        out_specs=pl.BlockSpec((TBLK, EMBED_DIM), lambda p: (p, 0)),
        out_shape=jax.ShapeDtypeStruct((B * S, EMBED_DIM), jnp.float32),
        scratch_shapes=[pltpu.VMEM((B * 2 * VPAD, EMBED_DIM), jnp.bfloat16)],
        compiler_params=pltpu.CompilerParams(
            dimension_semantics=("arbitrary",),
        ),
    )(
        tokens,
        table_pad,
        ln_gamma.reshape(1, EMBED_DIM),
        ln_beta.reshape(1, EMBED_DIM),
    )
    return out.reshape(B, S, EMBED_DIM)
